# baseline (speedup 1.0000x reference)
import jax
import jax.numpy as jnp
from jax import lax
from jax.experimental import pallas as pl
from jax.experimental.pallas import tpu as pltpu
from jax.experimental.pallas import tpu_sc as plsc

_NU = 1000000
_B = 16384
_NC = 2
_NS = 16
_NW = _NC * _NS
_PER_W = _B // _NW
_C = 256
_NCHUNK = _PER_W // _C
_D = 64


def _mf_body(z_hbm, uy_hbm, iy_hbm, u_hbm, i_hbm, r_hbm, out_hbm,
             uidx_v, iidx_v, zidx_v, r_v, zu_v, zi_v, yu_v, yi_v,
             acc_v, sem):
    wid = lax.axis_index("s") * _NC + lax.axis_index("c")
    base = wid * _PER_W
    acc = jnp.float32(0.0)

    for c in range(_NCHUNK):
        cbase = base + c * _C
        pltpu.sync_copy(u_hbm.at[pl.ds(cbase, _C)], uidx_v)
        pltpu.sync_copy(i_hbm.at[pl.ds(cbase, _C)], iidx_v)
        pltpu.sync_copy(r_hbm.at[pl.ds(cbase, _C)], r_v)
        for k in range(_C // 16):
            sl = pl.ds(k * 16, 16)
            zidx_v[sl] = iidx_v[sl] + _NU
        copies = [
            pltpu.async_copy(z_hbm.at[uidx_v], zu_v, sem),
            pltpu.async_copy(z_hbm.at[zidx_v], zi_v, sem),
            pltpu.async_copy(uy_hbm.at[uidx_v], yu_v, sem),
            pltpu.async_copy(iy_hbm.at[iidx_v], yi_v, sem),
        ]
        for cp in copies:
            cp.wait()

        def group(g, a):
            rv = r_v[pl.ds(g * 16, 16)]
            for k in range(16):
                j = g * 16 + k
                w = zu_v[j, pl.ds(0, 16)] * zi_v[j, pl.ds(0, 16)]
                for t in range(1, _D // 16):
                    sl = pl.ds(t * 16, 16)
                    w = w + zu_v[j, sl] * zi_v[j, sl]
                for t in range(_D // 16):
                    sl = pl.ds(t * 16, 16)
                    w = w + yu_v[j, sl] * yi_v[j, sl]
                e = rv[k] - jnp.sum(w)
                a = a + e * e
            return a

        acc = lax.fori_loop(0, _C // 16, group, acc)

    acc_v[:] = jnp.full((16,), 1.0, jnp.float32) * acc
    pltpu.sync_copy(acc_v, out_hbm.at[wid])


def kernel(Z, uY, iY, interaction):
    interaction = interaction.astype(jnp.int32)
    u = interaction[:, 0]
    i = interaction[:, 1]
    r = interaction[:, 2].astype(jnp.float32)
    f = pl.kernel(
        _mf_body,
        mesh=plsc.VectorSubcoreMesh(core_axis_name="c", subcore_axis_name="s"),
        compiler_params=pltpu.CompilerParams(
            needs_layout_passes=False, use_tc_tiling_on_sc=False),
        out_type=jax.ShapeDtypeStruct((_NW, 16), jnp.float32),
        scratch_types=[
            pltpu.VMEM((_C,), jnp.int32),
            pltpu.VMEM((_C,), jnp.int32),
            pltpu.VMEM((_C,), jnp.int32),
            pltpu.VMEM((_C,), jnp.float32),
            pltpu.VMEM((_C, _D), jnp.float32),
            pltpu.VMEM((_C, _D), jnp.float32),
            pltpu.VMEM((_C, _D), jnp.float32),
            pltpu.VMEM((_C, _D), jnp.float32),
            pltpu.VMEM((16,), jnp.float32),
            pltpu.SemaphoreType.DMA,
        ],
    )
    partial = f(Z, uY, iY, u, i, r)
    return jnp.sum(partial) / (_B * 16.0)


# speedup vs baseline: 2.4738x; 2.4738x over previous
import jax
import jax.numpy as jnp
from jax import lax
from jax.experimental import pallas as pl
from jax.experimental.pallas import tpu as pltpu
from jax.experimental.pallas import tpu_sc as plsc

_NU = 1000000
_NI = 100000
_B = 16384
_NC = 2
_NS = 16
_NW = _NC * _NS
_PER_W = _B // _NW
_CH = 16
_NCHUNK = _PER_W // _CH
_D = 64


def _mf_body(z_hbm, uy_hbm, iy_hbm, u_hbm, i_hbm, r_hbm, out_hbm,
             u_v, i_v, r_v, ut_v, us_v, zt_v, yt_v, is_v,
             zu_t, zi_t, yu_t, yi_t, acc_v, sem):
    wid = lax.axis_index("s") * _NC + lax.axis_index("c")
    base = wid * _PER_W

    pltpu.sync_copy(u_hbm.at[pl.ds(base, _PER_W)], u_v)
    pltpu.sync_copy(i_hbm.at[pl.ds(base, _PER_W)], i_v)
    pltpu.sync_copy(r_hbm.at[pl.ds(base, _PER_W)], r_v)

    for k in range(_PER_W // 16):
        sl = pl.ds(k * 16, 16)
        uu = u_v[sl]
        ii = i_v[sl]
        ut_v[sl] = lax.shift_right_logical(uu, 3)
        us_v[sl] = lax.bitwise_and(uu, 7)
        zt_v[sl] = lax.shift_right_logical(ii, 3) + (_NU // 8)
        yt_v[sl] = lax.shift_right_logical(ii, 3)
        is_v[sl] = lax.bitwise_and(ii, 7)

    def chunk(c, a):
        sl = pl.ds(c * _CH, _CH)
        ut_vec = ut_v[sl]
        us_vec = us_v[sl]
        zt_vec = zt_v[sl]
        yt_vec = yt_v[sl]
        is_vec = is_v[sl]
        copies = []
        for k in range(_CH):
            tu = ut_vec[k]
            su = us_vec[k]
            tz = zt_vec[k]
            ty = yt_vec[k]
            si = is_vec[k]
            copies.append(pltpu.async_copy(z_hbm.at[tu, su], zu_t.at[k], sem))
            copies.append(pltpu.async_copy(z_hbm.at[tz, si], zi_t.at[k], sem))
            copies.append(pltpu.async_copy(uy_hbm.at[tu, su], yu_t.at[k], sem))
            copies.append(pltpu.async_copy(iy_hbm.at[ty, si], yi_t.at[k], sem))
        for cp in copies:
            cp.wait()
        rv = r_v[sl]
        for k in range(_CH):
            w = zu_t[k, pl.ds(0, 16)] * zi_t[k, pl.ds(0, 16)]
            for t in range(1, _D // 16):
                ds = pl.ds(t * 16, 16)
                w = w + zu_t[k, ds] * zi_t[k, ds]
            for t in range(_D // 16):
                ds = pl.ds(t * 16, 16)
                w = w + yu_t[k, ds] * yi_t[k, ds]
            e = rv[k] - jnp.sum(w)
            a = a + e * e
        return a

    acc = lax.fori_loop(0, _NCHUNK, chunk, jnp.float32(0.0))

    acc_v[:] = jnp.full((16,), 1.0, jnp.float32) * acc
    pltpu.sync_copy(acc_v, out_hbm.at[wid])


def kernel(Z, uY, iY, interaction):
    interaction = interaction.astype(jnp.int32)
    u = interaction[:, 0]
    i = interaction[:, 1]
    r = interaction[:, 2].astype(jnp.float32)
    Zt = Z.reshape(-1, 8, _D)
    uYt = uY.reshape(-1, 8, _D)
    iYt = iY.reshape(-1, 8, _D)
    f = pl.kernel(
        _mf_body,
        mesh=plsc.VectorSubcoreMesh(core_axis_name="c", subcore_axis_name="s"),
        compiler_params=pltpu.CompilerParams(needs_layout_passes=False),
        out_type=jax.ShapeDtypeStruct((_NW, 16), jnp.float32),
        scratch_types=[
            pltpu.VMEM((_PER_W,), jnp.int32),
            pltpu.VMEM((_PER_W,), jnp.int32),
            pltpu.VMEM((_PER_W,), jnp.float32),
            pltpu.VMEM((_PER_W,), jnp.int32),
            pltpu.VMEM((_PER_W,), jnp.int32),
            pltpu.VMEM((_PER_W,), jnp.int32),
            pltpu.VMEM((_PER_W,), jnp.int32),
            pltpu.VMEM((_PER_W,), jnp.int32),
            pltpu.VMEM((_CH, _D), jnp.float32),
            pltpu.VMEM((_CH, _D), jnp.float32),
            pltpu.VMEM((_CH, _D), jnp.float32),
            pltpu.VMEM((_CH, _D), jnp.float32),
            pltpu.VMEM((16,), jnp.float32),
            pltpu.SemaphoreType.DMA,
        ],
    )
    partial = f(Zt, uYt, iYt, u, i, r)
    return jnp.sum(partial) / (_B * 16.0)
